# Initial kernel scaffold; baseline (speedup 1.0000x reference)
import jax
import jax.numpy as jnp
from jax.experimental import pallas as pl


def kernel(x, pos_edge_index, neg_edge_index, W1, b1, W2, b2):
    raise NotImplementedError("write your pallas kernel here")


# speedup vs baseline: 24.0790x; 24.0790x over previous
import functools

import jax
import jax.numpy as jnp
from jax import lax
from jax.experimental import pallas as pl
from jax.experimental.pallas import tpu as pltpu
from jax.experimental.pallas import tpu_sc as plsc

N = 10000
E = 320000
D = 128
H = 16
NC, NS = 2, 16
NW = NC * NS
NP = 10240
ROWS_PT = NP // NS
EPW = E // NW
CH = 80
NCHUNK = EPW // CH
DE = 2 * E
DEPW = DE // NW
NB16 = DEPW // 16
F_AUG = H + 2

_SC_MESH = plsc.VectorSubcoreMesh(
    core_axis_name="c", subcore_axis_name="s", num_cores=NC, num_subcores=NS)
_SC_PARAMS = pltpu.CompilerParams(use_tc_tiling_on_sc=False)


@functools.partial(
    pl.kernel,
    out_type=jax.ShapeDtypeStruct((NC, NS, ROWS_PT), jnp.float32),
    mesh=_SC_MESH,
    scratch_types=[
        pltpu.VMEM((NCHUNK, CH), jnp.int32),
        pltpu.VMEM((CH,), jnp.float32),
        pltpu.VMEM_SHARED((NP,), jnp.float32),
    ],
    compiler_params=_SC_PARAMS,
)
def _sc_deg(dst_hbm, zeros_hbm, deg_out, didx_v, ones_v, acc_sh):
    cid = lax.axis_index("c")
    sid = lax.axis_index("s")
    wid = cid * NS + sid
    pltpu.sync_copy(zeros_hbm, acc_sh.at[pl.ds(sid * ROWS_PT, ROWS_PT)])
    pltpu.sync_copy(dst_hbm.at[wid], didx_v)
    for i in range(CH // 16):
        ones_v[pl.ds(i * 16, 16)] = jnp.full((16,), 1.0, jnp.float32)
    plsc.subcore_barrier()

    def _chunk(j, carry):
        pltpu.sync_copy(ones_v, acc_sh.at[didx_v.at[j]], add=True)
        return carry

    lax.fori_loop(0, NCHUNK, _chunk, 0)
    plsc.subcore_barrier()
    pltpu.sync_copy(acc_sh.at[pl.ds(sid * ROWS_PT, ROWS_PT)], deg_out.at[cid, sid])


@functools.partial(
    pl.kernel,
    out_type=jax.ShapeDtypeStruct((NC, NS, ROWS_PT, H), jnp.float32),
    mesh=_SC_MESH,
    scratch_types=[
        pltpu.VMEM((NCHUNK, CH), jnp.int32),
        pltpu.VMEM((NCHUNK, CH), jnp.int32),
        pltpu.VMEM((CH, H), jnp.float32),
        pltpu.VMEM_SHARED((NP, H), jnp.float32),
        pltpu.SemaphoreType.DMA,
    ],
    compiler_params=_SC_PARAMS,
)
def _sc_conv(src_hbm, dst_hbm, table_hbm, zeros_hbm, acc_out,
             sidx_v, didx_v, msg_v, acc_sh, sem):
    cid = lax.axis_index("c")
    sid = lax.axis_index("s")
    wid = cid * NS + sid
    pltpu.sync_copy(zeros_hbm, acc_sh.at[pl.ds(sid * ROWS_PT, ROWS_PT)])
    pltpu.sync_copy(src_hbm.at[wid], sidx_v)
    pltpu.sync_copy(dst_hbm.at[wid], didx_v)
    plsc.subcore_barrier()

    def _chunk(j, carry):
        pltpu.async_copy(table_hbm.at[sidx_v.at[j]], msg_v, sem).wait()
        pltpu.sync_copy(msg_v, acc_sh.at[didx_v.at[j]], add=True)
        return carry

    lax.fori_loop(0, NCHUNK, _chunk, 0)
    plsc.subcore_barrier()
    pltpu.sync_copy(acc_sh.at[pl.ds(sid * ROWS_PT, ROWS_PT)], acc_out.at[cid, sid])


@functools.partial(
    pl.kernel,
    out_type=jax.ShapeDtypeStruct((NW, DEPW), jnp.float32),
    mesh=_SC_MESH,
    scratch_types=[
        pltpu.VMEM((DEPW,), jnp.int32),
        pltpu.VMEM((DEPW,), jnp.int32),
        pltpu.VMEM((DEPW,), jnp.float32),
        pltpu.VMEM((NP,), jnp.float32),
        pltpu.VMEM((NP,), jnp.float32),
    ],
    compiler_params=pltpu.CompilerParams(
        use_tc_tiling_on_sc=False, needs_layout_passes=False),
)
def _sc_dec(sidx_hbm, tidx_hbm, ut_hbm, vt_hbm, out_hbm,
            sv_v, tv_v, acc_v, urow_v, vrow_v):
    cid = lax.axis_index("c")
    sid = lax.axis_index("s")
    wid = cid * NS + sid
    pltpu.sync_copy(sidx_hbm.at[wid], sv_v)
    pltpu.sync_copy(tidx_hbm.at[wid], tv_v)

    def _zero(b, carry):
        acc_v[pl.ds(b * 16, 16)] = jnp.zeros((16,), jnp.float32)
        return carry

    lax.fori_loop(0, NB16, _zero, 0)
    for k in range(F_AUG):
        pltpu.sync_copy(ut_hbm.at[k], urow_v)
        pltpu.sync_copy(vt_hbm.at[k], vrow_v)

        def _dot(b, carry):
            s = sv_v[pl.ds(b * 16, 16)]
            t = tv_v[pl.ds(b * 16, 16)]
            a = plsc.load_gather(urow_v, [s])
            bb = plsc.load_gather(vrow_v, [t])
            acc_v[pl.ds(b * 16, 16)] = acc_v[pl.ds(b * 16, 16)] + a * bb
            return carry

        lax.fori_loop(0, NB16, _dot, 0)

    def _sig(b, carry):
        v = acc_v[pl.ds(b * 16, 16)]
        acc_v[pl.ds(b * 16, 16)] = 1.0 / (1.0 + jnp.exp(-v))
        return carry

    lax.fori_loop(0, NB16, _sig, 0)
    pltpu.sync_copy(acc_v, out_hbm.at[wid])


def _tc1_body(x_ref, w1_ref, degp_ref, hpT_ref, dinvT_ref):
    h0T = lax.dot_general(w1_ref[...], x_ref[...], (((0,), (1,)), ((), ())),
                          preferred_element_type=jnp.float32)
    deg = degp_ref[0:1, :] + degp_ref[1:2, :] + 1.0
    dinvT = lax.rsqrt(deg)
    hpT_ref[...] = h0T * dinvT
    dinvT_ref[...] = dinvT


_tc1 = pl.pallas_call(
    _tc1_body,
    out_shape=[jax.ShapeDtypeStruct((H, N), jnp.float32),
               jax.ShapeDtypeStruct((1, N), jnp.float32)],
)


def _tc2_body(accT_ref, hpT_ref, dinvT_ref, b1_ref, hp2T_ref):
    s = accT_ref[0] + accT_ref[1] + hpT_ref[...]
    h = jnp.maximum(dinvT_ref[...] * s + b1_ref[...], 0.0)
    hp2T_ref[...] = h * dinvT_ref[...]


_tc2 = pl.pallas_call(
    _tc2_body,
    out_shape=jax.ShapeDtypeStruct((H, N), jnp.float32),
)


def _tc3_body(accT_ref, hp2T_ref, dinvT_ref, w2_ref, b2_ref, ut_ref, vt_ref):
    g2T = dinvT_ref[...] * (accT_ref[0] + accT_ref[1] + hp2T_ref[...])
    w2 = w2_ref[...]
    q = lax.dot_general(w2, w2, (((1,), (1,)), ((), ())),
                        preferred_element_type=jnp.float32)
    pT = lax.dot_general(q, g2T, (((1,), (0,)), ((), ())),
                         preferred_element_type=jnp.float32)
    b2 = b2_ref[...]
    w2b2 = lax.dot_general(w2, b2, (((1,), (1,)), ((), ())),
                           preferred_element_type=jnp.float32)
    dT = lax.dot_general(w2b2, g2T, (((0,), (0,)), ((), ())),
                         preferred_element_type=jnp.float32)
    c = jnp.sum(b2 * b2)
    ones = jnp.ones((1, N), jnp.float32)
    ut_ref[...] = jnp.concatenate([g2T, dT, ones], axis=0)
    vt_ref[...] = jnp.concatenate([pT, ones, dT + c], axis=0)


_tc3 = pl.pallas_call(
    _tc3_body,
    out_shape=[jax.ShapeDtypeStruct((F_AUG, N), jnp.float32),
               jax.ShapeDtypeStruct((F_AUG, N), jnp.float32)],
)


def kernel(x, pos_edge_index, neg_edge_index, W1, b1, W2, b2):
    pei = pos_edge_index.astype(jnp.int32)
    nei = neg_edge_index.astype(jnp.int32)
    src_c = pei[0].reshape(NW, NCHUNK, CH)
    dst_c = pei[1].reshape(NW, NCHUNK, CH)
    deg_z = jnp.zeros((ROWS_PT,), jnp.float32)
    conv_z = jnp.zeros((ROWS_PT, H), jnp.float32)

    degp = _sc_deg(dst_c, deg_z)
    hpT, dinvT = _tc1(x, W1, degp.reshape(NC, NP)[:, :N])
    hp = jnp.pad(hpT.T, ((0, NP - N), (0, 0)))
    acc1 = _sc_conv(src_c, dst_c, hp, conv_z)
    acc1T = acc1.reshape(NC, NP, H)[:, :N, :].transpose(0, 2, 1)
    hp2T = _tc2(acc1T, hpT, dinvT, b1.reshape(H, 1))
    hp2 = jnp.pad(hp2T.T, ((0, NP - N), (0, 0)))
    acc2 = _sc_conv(src_c, dst_c, hp2, conv_z)
    acc2T = acc2.reshape(NC, NP, H)[:, :N, :].transpose(0, 2, 1)
    ut, vt = _tc3(acc2T, hp2T, dinvT, W2, b2.reshape(1, D))
    utp = jnp.pad(ut, ((0, 0), (0, NP - N)))
    vtp = jnp.pad(vt, ((0, 0), (0, NP - N)))
    s_all = jnp.concatenate([pei[0], nei[0]]).reshape(NW, DEPW)
    t_all = jnp.concatenate([pei[1], nei[1]]).reshape(NW, DEPW)
    logits = _sc_dec(s_all, t_all, utp, vtp)
    return logits.reshape(DE, 1)
